# baseline (speedup 1.0000x reference)
import functools

import jax
import jax.numpy as jnp
from jax import lax
from jax.experimental import pallas as pl
from jax.experimental.pallas import tpu as pltpu
from jax.experimental.pallas import tpu_sc as plsc

B = 16384
D = 64
NC = 2
NS = 16
NW = NC * NS
BPW = B // NW
L = 16


def _sc_body(uidx_hbm, pidx_hbm, uemb_hbm, pemb_hbm, ubias_hbm, pbias_hbm,
             out_hbm, uidx_v, pidx_v, urows_v, prows_v, ub_v, pb_v, out_v,
             sem):
    c = lax.axis_index("c")
    s = lax.axis_index("s")
    wid = s * NC + c
    base = wid * BPW

    pltpu.sync_copy(uidx_hbm.at[pl.ds(base, BPW)], uidx_v)
    pltpu.sync_copy(pidx_hbm.at[pl.ds(base, BPW)], pidx_v)

    cp1 = pltpu.async_copy(uemb_hbm.at[uidx_v], urows_v, sem)
    cp2 = pltpu.async_copy(pemb_hbm.at[pidx_v], prows_v, sem)
    cp3 = pltpu.async_copy(ubias_hbm.at[uidx_v], ub_v, sem)
    cp4 = pltpu.async_copy(pbias_hbm.at[pidx_v], pb_v, sem)
    cp1.wait()
    cp2.wait()
    cp3.wait()
    cp4.wait()

    last_lane = lax.iota(jnp.int32, L) == (L - 1)

    def body(r, carry):
        acc = urows_v[r, pl.ds(0, L)] * prows_v[r, pl.ds(0, L)]
        for k in range(1, D // L):
            acc = acc + urows_v[r, pl.ds(L * k, L)] * prows_v[r, pl.ds(L * k, L)]
        tot = plsc.cumsum(acc)
        plsc.store_scatter(out_v, [jnp.full((L,), r, jnp.int32)], tot,
                           mask=last_lane)
        return carry

    lax.fori_loop(0, BPW, body, 0)

    def bias_body(g, carry):
        sl = pl.ds(g * L, L)
        out_v[sl] = out_v[sl] + ub_v[sl] + pb_v[sl]
        return carry

    lax.fori_loop(0, BPW // L, bias_body, 0)
    pltpu.sync_copy(out_v, out_hbm.at[pl.ds(base, BPW)])


@jax.jit
def _run(uidx, pidx, user_emb, place_emb, user_bias, place_bias):
    mesh = plsc.VectorSubcoreMesh(core_axis_name="c", subcore_axis_name="s")
    kern = functools.partial(
        pl.kernel,
        out_type=jax.ShapeDtypeStruct((B,), jnp.float32),
        mesh=mesh,
        compiler_params=pltpu.CompilerParams(
            needs_layout_passes=False, use_tc_tiling_on_sc=False),
        scratch_types=[
            pltpu.VMEM((BPW,), jnp.int32),
            pltpu.VMEM((BPW,), jnp.int32),
            pltpu.VMEM((BPW, D), jnp.float32),
            pltpu.VMEM((BPW, D), jnp.float32),
            pltpu.VMEM((BPW,), jnp.float32),
            pltpu.VMEM((BPW,), jnp.float32),
            pltpu.VMEM((BPW,), jnp.float32),
            pltpu.SemaphoreType.DMA,
        ],
    )(_sc_body)
    return kern(uidx, pidx, user_emb, place_emb, user_bias, place_bias)


def kernel(inputs, user_emb, place_emb, user_bias, place_bias):
    uidx = inputs[:, 0]
    pidx = inputs[:, 1]
    out = _run(uidx, pidx, user_emb, place_emb,
               user_bias.reshape(-1), place_bias.reshape(-1))
    return out.reshape(B, 1)
